# Initial kernel scaffold; baseline (speedup 1.0000x reference)
import jax
import jax.numpy as jnp
from jax.experimental import pallas as pl


def kernel(tfidf_features, word_indices, geo_indices, tfidf_W1, tfidf_b1, tfidf_W2, tfidf_b2, word_emb, pos_emb, geo_emb, comp_W1, comp_b1, comp_W2, comp_b2, tfidf_scale, geo_scale):
    raise NotImplementedError("write your pallas kernel here")


# speedup vs baseline: 13.6645x; 13.6645x over previous
import functools
import math

import jax
import jax.numpy as jnp
from jax import lax
from jax.experimental import pallas as pl
from jax.experimental.pallas import tpu as pltpu
from jax.experimental.pallas import tpu_sc as plsc

B = 16384
L = 50
MAXF = 1000
NN = 256
D = NN // 4
GEO_PAD = 32

NW = 32
BPW = B // NW
R = 4
RL = R * L
NCH = BPW // R
INV_L = 1.0 / L


def _sc_body(emb_hbm, widx_hbm, gidx_hbm, wf_hbm, cnt_hbm,
             widx_v, gidx_v, buf0, buf1, ob0, ob1, cvm,
             sem0, sem1, osem0, osem1):
    nc = 2
    wid = lax.axis_index("s") * nc + lax.axis_index("c")
    base = wid * BPW

    pltpu.sync_copy(widx_hbm.at[pl.ds(base * L, BPW * L)], widx_v)
    pltpu.sync_copy(gidx_hbm.at[pl.ds(base * L, BPW * L)], gidx_v)

    def gather(c, buf, sem):
        return pltpu.make_async_copy(
            emb_hbm.at[widx_v.at[pl.ds(c * RL, RL)]], buf, sem)

    def outcp(c, ob, sem):
        return pltpu.make_async_copy(
            ob, wf_hbm.at[pl.ds(base + c * R, R)], sem)

    def reduce_chunk(buf, ob):
        for r in range(R):
            accs = [buf[r * L, pl.ds(16 * j, 16)] for j in range(4)]
            for l in range(1, L):
                for j in range(4):
                    accs[j] = accs[j] + buf[r * L + l, pl.ds(16 * j, 16)]
            for j in range(4):
                ob[r, pl.ds(16 * j, 16)] = accs[j] * INV_L

    gather(0, buf0, sem0).start()

    def pair_body(p, _):
        c0 = 2 * p
        gather(c0 + 1, buf1, sem1).start()
        gather(c0, buf0, sem0).wait()

        @pl.when(p > 0)
        def _():
            outcp(c0 - 2, ob0, osem0).wait()

        reduce_chunk(buf0, ob0)
        outcp(c0, ob0, osem0).start()

        @pl.when(p < NCH // 2 - 1)
        def _():
            gather(c0 + 2, buf0, sem0).start()

        gather(c0 + 1, buf1, sem1).wait()

        @pl.when(p > 0)
        def _():
            outcp(c0 - 1, ob1, osem1).wait()

        reduce_chunk(buf1, ob1)
        outcp(c0 + 1, ob1, osem1).start()
        return 0

    lax.fori_loop(0, NCH // 2, pair_body, 0)
    outcp(NCH - 2, ob0, osem0).wait()
    outcp(NCH - 1, ob1, osem1).wait()

    zero16 = jnp.zeros((16,), jnp.float32)
    ones16 = jnp.full((16,), INV_L, jnp.float32)
    iota16 = lax.iota(jnp.int32, 16)

    def geo_group(gi, _):
        for r in range(16):
            cvm[r, pl.ds(0, 16)] = zero16
            cvm[r, pl.ds(16, 16)] = zero16
        gbase = gi * 16 * L
        for l in range(L):
            vals = plsc.load_gather(gidx_v, [gbase + iota16 * L + l])
            plsc.addupdate_scatter(cvm, [iota16, vals], ones16)
        pltpu.sync_copy(cvm, cnt_hbm.at[pl.ds(base + gi * 16, 16)])
        return 0

    lax.fori_loop(0, BPW // 16, geo_group, 0)


_sc_gather = pl.kernel(
    _sc_body,
    out_type=(
        jax.ShapeDtypeStruct((B, D), jnp.float32),
        jax.ShapeDtypeStruct((B, GEO_PAD), jnp.float32),
    ),
    mesh=plsc.VectorSubcoreMesh(core_axis_name="c", subcore_axis_name="s"),
    compiler_params=pltpu.CompilerParams(
        needs_layout_passes=False, use_tc_tiling_on_sc=False),
    scratch_types=[
        pltpu.VMEM((BPW * L,), jnp.int32),
        pltpu.VMEM((BPW * L,), jnp.int32),
        pltpu.VMEM((RL, D), jnp.float32),
        pltpu.VMEM((RL, D), jnp.float32),
        pltpu.VMEM((R, D), jnp.float32),
        pltpu.VMEM((R, D), jnp.float32),
        pltpu.VMEM((16, GEO_PAD), jnp.float32),
        pltpu.SemaphoreType.DMA,
        pltpu.SemaphoreType.DMA,
        pltpu.SemaphoreType.DMA,
        pltpu.SemaphoreType.DMA,
    ],
)


BT = 512


def _dense_body(scales_ref, x_ref, wf_ref, cnt_ref, pos_ref,
                w1_ref, b1_ref, w2_ref, b2_ref, geo_ref,
                cw1a_ref, cw1b_ref, cw1c_ref, cw1d_ref, cb1_ref,
                cw2_ref, cb2_ref, o_ref):
    f32 = jnp.float32
    tf_scale = scales_ref[0, 0]
    geo_scale = scales_ref[0, 1]

    h = jnp.dot(x_ref[...], w1_ref[...], preferred_element_type=f32)
    h = h + b1_ref[...]
    h = jnp.dot(h, w2_ref[...], preferred_element_type=f32) + b2_ref[...]
    tf = jax.nn.sigmoid(jnp.clip(h * tf_scale, -10.0, 10.0))

    pos_row = jnp.sum(pos_ref[...], axis=0, keepdims=True) * INV_L

    c1 = jnp.dot(tf, cw1a_ref[...], preferred_element_type=f32)
    c1 = c1 + jnp.dot(wf_ref[...], cw1b_ref[...], preferred_element_type=f32)
    c1 = c1 + jnp.dot(pos_row, cw1c_ref[...], preferred_element_type=f32)
    geo_comb = jnp.dot(geo_ref[...], cw1d_ref[...], preferred_element_type=f32)
    c1 = c1 + jnp.dot(cnt_ref[...], geo_comb, preferred_element_type=f32)
    c1 = c1 + cb1_ref[...]

    c2 = jnp.dot(c1, cw2_ref[...], preferred_element_type=f32) + cb2_ref[...]
    compass = jax.nn.sigmoid(jnp.clip(c2 * geo_scale, -10.0, 10.0))
    gt = jnp.sin(compass * (math.pi / 4)) * jnp.cos(compass * (math.pi / 6))
    o_ref[...] = compass * 0.9 + gt * 0.1


def _dense(scales, x, wf, cnt, pos, w1, b1, w2, b2, geo,
           cw1a, cw1b, cw1c, cw1d, cb1, cw2, cb2):
    const = lambda shape: pl.BlockSpec(shape, lambda i: (0, 0))
    tiled = lambda shape: pl.BlockSpec(shape, lambda i: (i, 0))
    return pl.pallas_call(
        _dense_body,
        grid=(B // BT,),
        in_specs=[
            const((1, 2)),
            tiled((BT, MAXF)),
            tiled((BT, D)),
            tiled((BT, GEO_PAD)),
            const((L, D)),
            const((MAXF, D)),
            const((1, D)),
            const((D, D)),
            const((1, D)),
            const((GEO_PAD, D)),
            const((D, NN)),
            const((D, NN)),
            const((D, NN)),
            const((D, NN)),
            const((1, NN)),
            const((NN, NN)),
            const((1, NN)),
        ],
        out_specs=tiled((BT, NN)),
        out_shape=jax.ShapeDtypeStruct((B, NN), jnp.float32),
        compiler_params=pltpu.CompilerParams(
            dimension_semantics=("arbitrary",),
        ),
    )(scales, x, wf, cnt, pos, w1, b1, w2, b2, geo,
      cw1a, cw1b, cw1c, cw1d, cb1, cw2, cb2)


def kernel(tfidf_features, word_indices, geo_indices,
           tfidf_W1, tfidf_b1, tfidf_W2, tfidf_b2,
           word_emb, pos_emb, geo_emb,
           comp_W1, comp_b1, comp_W2, comp_b2,
           tfidf_scale, geo_scale):
    widx = word_indices.reshape(-1).astype(jnp.int32)
    gidx = geo_indices.reshape(-1).astype(jnp.int32)
    wf, cnt = _sc_gather(word_emb, widx, gidx)

    scales = jnp.stack([tfidf_scale, geo_scale]).reshape(1, 2).astype(jnp.float32)
    pos = pos_emb[:L]
    geo = geo_emb[:GEO_PAD]
    cw1a = comp_W1[0:D]
    cw1b = comp_W1[D:2 * D]
    cw1c = comp_W1[2 * D:3 * D]
    cw1d = comp_W1[3 * D:4 * D]
    return _dense(scales, tfidf_features, wf, cnt, pos,
                  tfidf_W1, tfidf_b1.reshape(1, D), tfidf_W2,
                  tfidf_b2.reshape(1, D), geo,
                  cw1a, cw1b, cw1c, cw1d, comp_b1.reshape(1, NN),
                  comp_W2, comp_b2.reshape(1, NN))


# speedup vs baseline: 16.3565x; 1.1970x over previous
import functools
import math

import jax
import jax.numpy as jnp
from jax import lax
from jax.experimental import pallas as pl
from jax.experimental.pallas import tpu as pltpu
from jax.experimental.pallas import tpu_sc as plsc

B = 16384
L = 50
MAXF = 1000
NN = 256
D = NN // 4
GEO_PAD = 32

NW = 32
BPW = B // NW
R = 8
RL = R * L
NCH = BPW // R
INV_L = 1.0 / L


def _sc_body(emb_hbm, widx_hbm, gidx_hbm, wf_hbm, cnt_hbm,
             widx_v, gidx_v, buf0, buf1, ob0, ob1, cvm,
             sem0, sem1, osem0, osem1):
    nc = 2
    wid = lax.axis_index("s") * nc + lax.axis_index("c")
    base = wid * BPW

    pltpu.sync_copy(widx_hbm.at[pl.ds(base * L, BPW * L)], widx_v)
    pltpu.sync_copy(gidx_hbm.at[pl.ds(base * L, BPW * L)], gidx_v)

    def gather(c, buf, sem):
        return pltpu.make_async_copy(
            emb_hbm.at[widx_v.at[pl.ds(c * RL, RL)]], buf, sem)

    def outcp(c, ob, sem):
        return pltpu.make_async_copy(
            ob, wf_hbm.at[pl.ds(base + c * R, R)], sem)

    inv_l = jnp.full((32,), INV_L, jnp.bfloat16)

    def reduce_chunk(buf, ob):
        for r in range(R):
            acc = [[buf[r * L + p, pl.ds(32 * j, 32)] for j in range(2)]
                   for p in range(2)]
            for l in range(2, L):
                p = l % 2
                for j in range(2):
                    acc[p][j] = acc[p][j] + buf[r * L + l, pl.ds(32 * j, 32)]
            for j in range(2):
                ob[r, pl.ds(32 * j, 32)] = (acc[0][j] + acc[1][j]) * inv_l

    gather(0, buf0, sem0).start()

    def pair_body(p, _):
        c0 = 2 * p
        gather(c0 + 1, buf1, sem1).start()
        gather(c0, buf0, sem0).wait()

        @pl.when(p > 0)
        def _():
            outcp(c0 - 2, ob0, osem0).wait()

        reduce_chunk(buf0, ob0)
        outcp(c0, ob0, osem0).start()

        @pl.when(p < NCH // 2 - 1)
        def _():
            gather(c0 + 2, buf0, sem0).start()

        gather(c0 + 1, buf1, sem1).wait()

        @pl.when(p > 0)
        def _():
            outcp(c0 - 1, ob1, osem1).wait()

        reduce_chunk(buf1, ob1)
        outcp(c0 + 1, ob1, osem1).start()
        return 0

    lax.fori_loop(0, NCH // 2, pair_body, 0)
    outcp(NCH - 2, ob0, osem0).wait()
    outcp(NCH - 1, ob1, osem1).wait()

    zero16 = jnp.zeros((16,), jnp.float32)
    ones16 = jnp.full((16,), INV_L, jnp.float32)
    iota16 = lax.iota(jnp.int32, 16)

    def geo_group(gi, _):
        for r in range(16):
            cvm[r, pl.ds(0, 16)] = zero16
            cvm[r, pl.ds(16, 16)] = zero16
        gbase = gi * 16 * L
        for l in range(L):
            vals = plsc.load_gather(gidx_v, [gbase + iota16 * L + l])
            plsc.addupdate_scatter(cvm, [iota16, vals], ones16)
        pltpu.sync_copy(cvm, cnt_hbm.at[pl.ds(base + gi * 16, 16)])
        return 0

    lax.fori_loop(0, BPW // 16, geo_group, 0)


_sc_gather = pl.kernel(
    _sc_body,
    out_type=(
        jax.ShapeDtypeStruct((B, D), jnp.bfloat16),
        jax.ShapeDtypeStruct((B, GEO_PAD), jnp.float32),
    ),
    mesh=plsc.VectorSubcoreMesh(core_axis_name="c", subcore_axis_name="s"),
    compiler_params=pltpu.CompilerParams(
        needs_layout_passes=False, use_tc_tiling_on_sc=False),
    scratch_types=[
        pltpu.VMEM((BPW * L,), jnp.int32),
        pltpu.VMEM((BPW * L,), jnp.int32),
        pltpu.VMEM((RL, D), jnp.bfloat16),
        pltpu.VMEM((RL, D), jnp.bfloat16),
        pltpu.VMEM((R, D), jnp.bfloat16),
        pltpu.VMEM((R, D), jnp.bfloat16),
        pltpu.VMEM((16, GEO_PAD), jnp.float32),
        pltpu.SemaphoreType.DMA,
        pltpu.SemaphoreType.DMA,
        pltpu.SemaphoreType.DMA,
        pltpu.SemaphoreType.DMA,
    ],
)


BT = 512


def _dense_body(scales_ref, x_ref, wf_ref, cnt_ref, pos_ref,
                w1_ref, b1_ref, w2_ref, b2_ref, geo_ref,
                cw1a_ref, cw1b_ref, cw1c_ref, cw1d_ref, cb1_ref,
                cw2_ref, cb2_ref, o_ref):
    f32 = jnp.float32
    bf16 = jnp.bfloat16
    tf_scale = scales_ref[0, 0]
    geo_scale = scales_ref[0, 1]

    h = jnp.dot(x_ref[...], w1_ref[...], preferred_element_type=f32)
    h = h + b1_ref[...]
    h = jnp.dot(h.astype(bf16), w2_ref[...],
                preferred_element_type=f32) + b2_ref[...]
    tf = jax.nn.sigmoid(jnp.clip(h * tf_scale, -10.0, 10.0))

    pos_row = jnp.sum(pos_ref[...], axis=0, keepdims=True) * INV_L

    c1 = jnp.dot(tf.astype(bf16), cw1a_ref[...], preferred_element_type=f32)
    c1 = c1 + jnp.dot(wf_ref[...], cw1b_ref[...], preferred_element_type=f32)
    c1 = c1 + jnp.dot(pos_row.astype(bf16), cw1c_ref[...],
                      preferred_element_type=f32)
    geo_comb = jnp.dot(geo_ref[...], cw1d_ref[...], preferred_element_type=f32)
    c1 = c1 + jnp.dot(cnt_ref[...].astype(bf16), geo_comb.astype(bf16),
                      preferred_element_type=f32)
    c1 = c1 + cb1_ref[...]

    c2 = jnp.dot(c1.astype(bf16), cw2_ref[...],
                 preferred_element_type=f32) + cb2_ref[...]
    compass = jax.nn.sigmoid(jnp.clip(c2 * geo_scale, -10.0, 10.0))
    gt = jnp.sin(compass * (math.pi / 4)) * jnp.cos(compass * (math.pi / 6))
    o_ref[...] = compass * 0.9 + gt * 0.1


def _dense(scales, x, wf, cnt, pos, w1, b1, w2, b2, geo,
           cw1a, cw1b, cw1c, cw1d, cb1, cw2, cb2):
    const = lambda shape: pl.BlockSpec(shape, lambda i: (0, 0))
    tiled = lambda shape: pl.BlockSpec(shape, lambda i: (i, 0))
    return pl.pallas_call(
        _dense_body,
        grid=(B // BT,),
        in_specs=[
            const((1, 2)),
            tiled((BT, MAXF)),
            tiled((BT, D)),
            tiled((BT, GEO_PAD)),
            const((L, D)),
            const((MAXF, D)),
            const((1, D)),
            const((D, D)),
            const((1, D)),
            const((GEO_PAD, D)),
            const((D, NN)),
            const((D, NN)),
            const((D, NN)),
            const((D, NN)),
            const((1, NN)),
            const((NN, NN)),
            const((1, NN)),
        ],
        out_specs=tiled((BT, NN)),
        out_shape=jax.ShapeDtypeStruct((B, NN), jnp.float32),
        compiler_params=pltpu.CompilerParams(
            dimension_semantics=("arbitrary",),
        ),
    )(scales, x, wf, cnt, pos, w1, b1, w2, b2, geo,
      cw1a, cw1b, cw1c, cw1d, cb1, cw2, cb2)


def kernel(tfidf_features, word_indices, geo_indices,
           tfidf_W1, tfidf_b1, tfidf_W2, tfidf_b2,
           word_emb, pos_emb, geo_emb,
           comp_W1, comp_b1, comp_W2, comp_b2,
           tfidf_scale, geo_scale):
    bf16 = jnp.bfloat16
    widx = word_indices.reshape(-1).astype(jnp.int32)
    gidx = geo_indices.reshape(-1).astype(jnp.int32)
    wf, cnt = _sc_gather(word_emb.astype(bf16), widx, gidx)

    scales = jnp.stack([tfidf_scale, geo_scale]).reshape(1, 2).astype(jnp.float32)
    pos = pos_emb[:L]
    geo = geo_emb[:GEO_PAD].astype(bf16)
    cw1a = comp_W1[0:D].astype(bf16)
    cw1b = comp_W1[D:2 * D].astype(bf16)
    cw1c = comp_W1[2 * D:3 * D].astype(bf16)
    cw1d = comp_W1[3 * D:4 * D].astype(bf16)
    return _dense(scales, tfidf_features.astype(bf16), wf, cnt, pos,
                  tfidf_W1.astype(bf16), tfidf_b1.reshape(1, D),
                  tfidf_W2.astype(bf16),
                  tfidf_b2.reshape(1, D), geo,
                  cw1a, cw1b, cw1c, cw1d, comp_b1.reshape(1, NN),
                  comp_W2.astype(bf16), comp_b2.reshape(1, NN))


# speedup vs baseline: 20.6844x; 1.2646x over previous
import functools
import math

import jax
import jax.numpy as jnp
from jax import lax
from jax.experimental import pallas as pl
from jax.experimental.pallas import tpu as pltpu
from jax.experimental.pallas import tpu_sc as plsc

B = 16384
L = 50
MAXF = 1000
NN = 256
D = NN // 4
GEO_PAD = 32

NW = 32
BPW = B // NW
R = 8
RL = R * L
NCH = BPW // R
INV_L = 1.0 / L


def _sc_body(emb_hbm, widx_hbm, gidx_hbm, wf_hbm, cnt_hbm,
             widx_v, gidx_v, buf0, buf1, ob0, ob1, cvm,
             sem0, sem1, osem0, osem1):
    nc = 2
    wid = lax.axis_index("s") * nc + lax.axis_index("c")
    base = wid * BPW

    pltpu.sync_copy(widx_hbm.at[pl.ds(base * L, BPW * L)], widx_v)
    pltpu.sync_copy(gidx_hbm.at[pl.ds(base * L, BPW * L)], gidx_v)

    def gather(c, buf, sem):
        return pltpu.make_async_copy(
            emb_hbm.at[widx_v.at[pl.ds(c * RL, RL)]], buf, sem)

    def outcp(c, ob, sem):
        return pltpu.make_async_copy(
            ob, wf_hbm.at[pl.ds(base + c * R, R)], sem)

    inv_l = jnp.full((32,), INV_L, jnp.bfloat16)

    def reduce_chunk(buf, ob):
        for r in range(R):
            acc = [[buf[r * L + p, pl.ds(32 * j, 32)] for j in range(2)]
                   for p in range(2)]
            for l in range(2, L):
                p = l % 2
                for j in range(2):
                    acc[p][j] = acc[p][j] + buf[r * L + l, pl.ds(32 * j, 32)]
            for j in range(2):
                ob[r, pl.ds(32 * j, 32)] = (acc[0][j] + acc[1][j]) * inv_l

    gather(0, buf0, sem0).start()

    def pair_body(p, _):
        c0 = 2 * p
        gather(c0 + 1, buf1, sem1).start()
        gather(c0, buf0, sem0).wait()

        @pl.when(p > 0)
        def _():
            outcp(c0 - 2, ob0, osem0).wait()

        reduce_chunk(buf0, ob0)
        outcp(c0, ob0, osem0).start()

        @pl.when(p < NCH // 2 - 1)
        def _():
            gather(c0 + 2, buf0, sem0).start()

        gather(c0 + 1, buf1, sem1).wait()

        @pl.when(p > 0)
        def _():
            outcp(c0 - 1, ob1, osem1).wait()

        reduce_chunk(buf1, ob1)
        outcp(c0 + 1, ob1, osem1).start()
        return 0

    lax.fori_loop(0, NCH // 2, pair_body, 0)
    outcp(NCH - 2, ob0, osem0).wait()
    outcp(NCH - 1, ob1, osem1).wait()

    zero16 = jnp.zeros((16,), jnp.float32)
    ones16 = jnp.full((16,), INV_L, jnp.float32)
    iota16 = lax.iota(jnp.int32, 16)

    def geo_group(gi, _):
        for r in range(16):
            cvm[r, pl.ds(0, 16)] = zero16
            cvm[r, pl.ds(16, 16)] = zero16
        gbase = gi * 16 * L
        for l in range(L):
            vals = plsc.load_gather(gidx_v, [gbase + iota16 * L + l])
            plsc.addupdate_scatter(cvm, [iota16, vals], ones16)
        pltpu.sync_copy(cvm, cnt_hbm.at[pl.ds(base + gi * 16, 16)])
        return 0

    lax.fori_loop(0, BPW // 16, geo_group, 0)


_sc_gather = pl.kernel(
    _sc_body,
    out_type=(
        jax.ShapeDtypeStruct((B, D), jnp.bfloat16),
        jax.ShapeDtypeStruct((B, GEO_PAD), jnp.float32),
    ),
    mesh=plsc.VectorSubcoreMesh(core_axis_name="c", subcore_axis_name="s"),
    compiler_params=pltpu.CompilerParams(
        needs_layout_passes=False, use_tc_tiling_on_sc=False),
    scratch_types=[
        pltpu.VMEM((BPW * L,), jnp.int32),
        pltpu.VMEM((BPW * L,), jnp.int32),
        pltpu.VMEM((RL, D), jnp.bfloat16),
        pltpu.VMEM((RL, D), jnp.bfloat16),
        pltpu.VMEM((R, D), jnp.bfloat16),
        pltpu.VMEM((R, D), jnp.bfloat16),
        pltpu.VMEM((16, GEO_PAD), jnp.float32),
        pltpu.SemaphoreType.DMA,
        pltpu.SemaphoreType.DMA,
        pltpu.SemaphoreType.DMA,
        pltpu.SemaphoreType.DMA,
    ],
)


BT = 512


def _dense_body(scales_ref, x_ref, wf_ref, cnt_ref, pos_ref,
                w1_ref, b1_ref, w2_ref, b2_ref, geo_ref,
                cw1a_ref, cw1b_ref, cw1c_ref, cw1d_ref, cb1_ref,
                cw2_ref, cb2_ref, o_ref):
    f32 = jnp.float32
    bf16 = jnp.bfloat16
    tf_scale = scales_ref[0, 0]
    geo_scale = scales_ref[0, 1]

    h = jnp.dot(x_ref[...].astype(bf16), w1_ref[...],
                preferred_element_type=f32)
    h = h + b1_ref[...]
    h = jnp.dot(h.astype(bf16), w2_ref[...],
                preferred_element_type=f32) + b2_ref[...]
    tf = jax.nn.sigmoid(jnp.clip(h * tf_scale, -10.0, 10.0))

    pos_row = jnp.sum(pos_ref[...], axis=0, keepdims=True) * INV_L

    c1 = jnp.dot(tf.astype(bf16), cw1a_ref[...], preferred_element_type=f32)
    c1 = c1 + jnp.dot(wf_ref[...], cw1b_ref[...], preferred_element_type=f32)
    c1 = c1 + jnp.dot(pos_row.astype(bf16), cw1c_ref[...],
                      preferred_element_type=f32)
    geo_comb = jnp.dot(geo_ref[...], cw1d_ref[...], preferred_element_type=f32)
    c1 = c1 + jnp.dot(cnt_ref[...].astype(bf16), geo_comb.astype(bf16),
                      preferred_element_type=f32)
    c1 = c1 + cb1_ref[...]

    c2 = jnp.dot(c1.astype(bf16), cw2_ref[...],
                 preferred_element_type=f32) + cb2_ref[...]
    compass = jax.nn.sigmoid(jnp.clip(c2 * geo_scale, -10.0, 10.0))
    a = compass * (math.pi / 4)
    b = compass * (math.pi / 6)
    a2 = a * a
    b2 = b * b
    sin_a = a * (1.0 + a2 * (-1.0 / 6.0 + a2 * (1.0 / 120.0
                                                + a2 * (-1.0 / 5040.0))))
    cos_b = 1.0 + b2 * (-0.5 + b2 * (1.0 / 24.0 + b2 * (-1.0 / 720.0)))
    o_ref[...] = compass * 0.9 + sin_a * cos_b * 0.1


def _dense(scales, x, wf, cnt, pos, w1, b1, w2, b2, geo,
           cw1a, cw1b, cw1c, cw1d, cb1, cw2, cb2):
    const = lambda shape: pl.BlockSpec(shape, lambda i: (0, 0))
    tiled = lambda shape: pl.BlockSpec(shape, lambda i: (i, 0))
    return pl.pallas_call(
        _dense_body,
        grid=(B // BT,),
        in_specs=[
            const((1, 2)),
            tiled((BT, MAXF)),
            tiled((BT, D)),
            tiled((BT, GEO_PAD)),
            const((L, D)),
            const((MAXF, D)),
            const((1, D)),
            const((D, D)),
            const((1, D)),
            const((GEO_PAD, D)),
            const((D, NN)),
            const((D, NN)),
            const((D, NN)),
            const((D, NN)),
            const((1, NN)),
            const((NN, NN)),
            const((1, NN)),
        ],
        out_specs=tiled((BT, NN)),
        out_shape=jax.ShapeDtypeStruct((B, NN), jnp.float32),
        compiler_params=pltpu.CompilerParams(
            dimension_semantics=("arbitrary",),
        ),
    )(scales, x, wf, cnt, pos, w1, b1, w2, b2, geo,
      cw1a, cw1b, cw1c, cw1d, cb1, cw2, cb2)


def kernel(tfidf_features, word_indices, geo_indices,
           tfidf_W1, tfidf_b1, tfidf_W2, tfidf_b2,
           word_emb, pos_emb, geo_emb,
           comp_W1, comp_b1, comp_W2, comp_b2,
           tfidf_scale, geo_scale):
    bf16 = jnp.bfloat16
    widx = word_indices.reshape(-1).astype(jnp.int32)
    gidx = geo_indices.reshape(-1).astype(jnp.int32)
    wf, cnt = _sc_gather(word_emb.astype(bf16), widx, gidx)

    scales = jnp.stack([tfidf_scale, geo_scale]).reshape(1, 2).astype(jnp.float32)
    pos = pos_emb[:L]
    geo = geo_emb[:GEO_PAD].astype(bf16)
    cw1a = comp_W1[0:D].astype(bf16)
    cw1b = comp_W1[D:2 * D].astype(bf16)
    cw1c = comp_W1[2 * D:3 * D].astype(bf16)
    cw1d = comp_W1[3 * D:4 * D].astype(bf16)
    return _dense(scales, tfidf_features, wf, cnt, pos,
                  tfidf_W1.astype(bf16), tfidf_b1.reshape(1, D),
                  tfidf_W2.astype(bf16),
                  tfidf_b2.reshape(1, D), geo,
                  cw1a, cw1b, cw1c, cw1d, comp_b1.reshape(1, NN),
                  comp_W2.astype(bf16), comp_b2.reshape(1, NN))


# speedup vs baseline: 24.3681x; 1.1781x over previous
import functools
import math

import jax
import jax.numpy as jnp
from jax import lax
from jax.experimental import pallas as pl
from jax.experimental.pallas import tpu as pltpu
from jax.experimental.pallas import tpu_sc as plsc

B = 16384
L = 50
MAXF = 1000
NN = 256
D = NN // 4
GEO_PAD = 32

NW = 32
BPW = B // NW
R = 16
RL = R * L
NCH = BPW // R
INV_L = 1.0 / L


def _sc_body(emb_hbm, widx_hbm, gidx_hbm, wf_hbm, cnt_hbm,
             widx_v, gidx_v, buf0, buf1, ob0, ob1, cvm,
             sem0, sem1, osem0, osem1):
    nc = 2
    wid = lax.axis_index("s") * nc + lax.axis_index("c")
    base = wid * BPW

    pltpu.sync_copy(widx_hbm.at[pl.ds(base * L, BPW * L)], widx_v)
    pltpu.sync_copy(gidx_hbm.at[pl.ds(base * L, BPW * L)], gidx_v)

    def gather(c, buf, sem):
        return pltpu.make_async_copy(
            emb_hbm.at[widx_v.at[pl.ds(c * RL, RL)]], buf, sem)

    def outcp(c, ob, sem):
        return pltpu.make_async_copy(
            ob, wf_hbm.at[pl.ds(base + c * R, R)], sem)

    inv_l = jnp.full((32,), INV_L, jnp.bfloat16)

    def reduce_chunk(buf, ob):
        def row_body(r, _):
            rb = r * L
            acc = [[buf[rb + p, pl.ds(32 * j, 32)] for j in range(2)]
                   for p in range(2)]
            for l in range(2, L):
                p = l % 2
                for j in range(2):
                    acc[p][j] = acc[p][j] + buf[rb + l, pl.ds(32 * j, 32)]
            for j in range(2):
                ob[r, pl.ds(32 * j, 32)] = (acc[0][j] + acc[1][j]) * inv_l
            return 0

        lax.fori_loop(0, R, row_body, 0)

    gather(0, buf0, sem0).start()

    def pair_body(p, _):
        c0 = 2 * p
        gather(c0 + 1, buf1, sem1).start()
        gather(c0, buf0, sem0).wait()

        @pl.when(p > 0)
        def _():
            outcp(c0 - 2, ob0, osem0).wait()

        reduce_chunk(buf0, ob0)
        outcp(c0, ob0, osem0).start()

        @pl.when(p < NCH // 2 - 1)
        def _():
            gather(c0 + 2, buf0, sem0).start()

        gather(c0 + 1, buf1, sem1).wait()

        @pl.when(p > 0)
        def _():
            outcp(c0 - 1, ob1, osem1).wait()

        reduce_chunk(buf1, ob1)
        outcp(c0 + 1, ob1, osem1).start()
        return 0

    lax.fori_loop(0, NCH // 2, pair_body, 0)
    outcp(NCH - 2, ob0, osem0).wait()
    outcp(NCH - 1, ob1, osem1).wait()

    zero16 = jnp.zeros((16,), jnp.float32)
    ones16 = jnp.full((16,), INV_L, jnp.float32)
    iota16 = lax.iota(jnp.int32, 16)

    def geo_group(gi, _):
        for r in range(16):
            cvm[r, pl.ds(0, 16)] = zero16
            cvm[r, pl.ds(16, 16)] = zero16
        gbase = gi * 16 * L
        for l in range(L):
            vals = plsc.load_gather(gidx_v, [gbase + iota16 * L + l])
            plsc.addupdate_scatter(cvm, [iota16, vals], ones16)
        pltpu.sync_copy(cvm, cnt_hbm.at[pl.ds(base + gi * 16, 16)])
        return 0

    lax.fori_loop(0, BPW // 16, geo_group, 0)


_sc_gather = pl.kernel(
    _sc_body,
    out_type=(
        jax.ShapeDtypeStruct((B, D), jnp.bfloat16),
        jax.ShapeDtypeStruct((B, GEO_PAD), jnp.float32),
    ),
    mesh=plsc.VectorSubcoreMesh(core_axis_name="c", subcore_axis_name="s"),
    compiler_params=pltpu.CompilerParams(
        needs_layout_passes=False, use_tc_tiling_on_sc=False),
    scratch_types=[
        pltpu.VMEM((BPW * L,), jnp.int32),
        pltpu.VMEM((BPW * L,), jnp.int32),
        pltpu.VMEM((RL, D), jnp.bfloat16),
        pltpu.VMEM((RL, D), jnp.bfloat16),
        pltpu.VMEM((R, D), jnp.bfloat16),
        pltpu.VMEM((R, D), jnp.bfloat16),
        pltpu.VMEM((16, GEO_PAD), jnp.float32),
        pltpu.SemaphoreType.DMA,
        pltpu.SemaphoreType.DMA,
        pltpu.SemaphoreType.DMA,
        pltpu.SemaphoreType.DMA,
    ],
)


BT = 512


def _const(shape):
    return pl.BlockSpec(shape, lambda i: (0, 0))


def _tiled(shape):
    return pl.BlockSpec(shape, lambda i: (i, 0))


def _tfidf_body(scales_ref, x_ref, w1_ref, b1_ref, w2_ref, b2_ref, tf_ref):
    f32 = jnp.float32
    bf16 = jnp.bfloat16
    tf_scale = scales_ref[0, 0]
    h = jnp.dot(x_ref[...].astype(bf16), w1_ref[...],
                preferred_element_type=f32)
    h = h + b1_ref[...]
    h = jnp.dot(h.astype(bf16), w2_ref[...],
                preferred_element_type=f32) + b2_ref[...]
    tf = jax.nn.sigmoid(jnp.clip(h * tf_scale, -10.0, 10.0))
    tf_ref[...] = tf.astype(bf16)


def _tfidf(scales, x, w1, b1, w2, b2):
    return pl.pallas_call(
        _tfidf_body,
        grid=(B // BT,),
        in_specs=[
            _const((1, 2)),
            _tiled((BT, MAXF)),
            _const((MAXF, D)),
            _const((1, D)),
            _const((D, D)),
            _const((1, D)),
        ],
        out_specs=_tiled((BT, D)),
        out_shape=jax.ShapeDtypeStruct((B, D), jnp.bfloat16),
        compiler_params=pltpu.CompilerParams(
            dimension_semantics=("arbitrary",),
        ),
    )(scales, x, w1, b1, w2, b2)


def _combine_body(scales_ref, tf_ref, wf_ref, cnt_ref, pos_ref, geo_ref,
                  cw1a_ref, cw1b_ref, cw1c_ref, cw1d_ref, cb1_ref,
                  cw2_ref, cb2_ref, o_ref):
    f32 = jnp.float32
    bf16 = jnp.bfloat16
    geo_scale = scales_ref[0, 1]

    pos_row = jnp.sum(pos_ref[...], axis=0, keepdims=True) * INV_L

    c1 = jnp.dot(tf_ref[...], cw1a_ref[...], preferred_element_type=f32)
    c1 = c1 + jnp.dot(wf_ref[...], cw1b_ref[...], preferred_element_type=f32)
    c1 = c1 + jnp.dot(pos_row.astype(bf16), cw1c_ref[...],
                      preferred_element_type=f32)
    geo_comb = jnp.dot(geo_ref[...], cw1d_ref[...], preferred_element_type=f32)
    c1 = c1 + jnp.dot(cnt_ref[...].astype(bf16), geo_comb.astype(bf16),
                      preferred_element_type=f32)
    c1 = c1 + cb1_ref[...]

    c2 = jnp.dot(c1.astype(bf16), cw2_ref[...],
                 preferred_element_type=f32) + cb2_ref[...]
    compass = jax.nn.sigmoid(jnp.clip(c2 * geo_scale, -10.0, 10.0))
    a = compass * (math.pi / 4)
    b = compass * (math.pi / 6)
    a2 = a * a
    b2 = b * b
    sin_a = a * (1.0 + a2 * (-1.0 / 6.0 + a2 * (1.0 / 120.0
                                                + a2 * (-1.0 / 5040.0))))
    cos_b = 1.0 + b2 * (-0.5 + b2 * (1.0 / 24.0 + b2 * (-1.0 / 720.0)))
    o_ref[...] = compass * 0.9 + sin_a * cos_b * 0.1


def _combine(scales, tf, wf, cnt, pos, geo,
             cw1a, cw1b, cw1c, cw1d, cb1, cw2, cb2):
    return pl.pallas_call(
        _combine_body,
        grid=(B // BT,),
        in_specs=[
            _const((1, 2)),
            _tiled((BT, D)),
            _tiled((BT, D)),
            _tiled((BT, GEO_PAD)),
            _const((L, D)),
            _const((GEO_PAD, D)),
            _const((D, NN)),
            _const((D, NN)),
            _const((D, NN)),
            _const((D, NN)),
            _const((1, NN)),
            _const((NN, NN)),
            _const((1, NN)),
        ],
        out_specs=_tiled((BT, NN)),
        out_shape=jax.ShapeDtypeStruct((B, NN), jnp.float32),
        compiler_params=pltpu.CompilerParams(
            dimension_semantics=("arbitrary",),
        ),
    )(scales, tf, wf, cnt, pos, geo,
      cw1a, cw1b, cw1c, cw1d, cb1, cw2, cb2)


def kernel(tfidf_features, word_indices, geo_indices,
           tfidf_W1, tfidf_b1, tfidf_W2, tfidf_b2,
           word_emb, pos_emb, geo_emb,
           comp_W1, comp_b1, comp_W2, comp_b2,
           tfidf_scale, geo_scale):
    bf16 = jnp.bfloat16
    widx = word_indices.reshape(-1).astype(jnp.int32)
    gidx = geo_indices.reshape(-1).astype(jnp.int32)
    wf, cnt = _sc_gather(word_emb.astype(bf16), widx, gidx)

    scales = jnp.stack([tfidf_scale, geo_scale]).reshape(1, 2).astype(jnp.float32)
    pos = pos_emb[:L]
    geo = geo_emb[:GEO_PAD].astype(bf16)
    cw1a = comp_W1[0:D].astype(bf16)
    cw1b = comp_W1[D:2 * D].astype(bf16)
    cw1c = comp_W1[2 * D:3 * D].astype(bf16)
    cw1d = comp_W1[3 * D:4 * D].astype(bf16)
    tf = _tfidf(scales, tfidf_features, tfidf_W1.astype(bf16),
                tfidf_b1.reshape(1, D), tfidf_W2.astype(bf16),
                tfidf_b2.reshape(1, D))
    return _combine(scales, tf, wf, cnt, pos, geo,
                    cw1a, cw1b, cw1c, cw1d, comp_b1.reshape(1, NN),
                    comp_W2.astype(bf16), comp_b2.reshape(1, NN))


# speedup vs baseline: 26.3738x; 1.0823x over previous
import functools
import math

import jax
import jax.numpy as jnp
from jax import lax
from jax.experimental import pallas as pl
from jax.experimental.pallas import tpu as pltpu
from jax.experimental.pallas import tpu_sc as plsc

B = 16384
L = 50
MAXF = 1000
NN = 256
D = NN // 4
GEO_PAD = 32
VOCAB_ROWS = 100001

NW = 32
BPW = B // NW
R = 16
RL = R * L
NCH = BPW // R
INV_L = 1.0 / L


def _sc_body(emb_hbm, widx_hbm, gidx_hbm, wf_hbm, cnt_hbm,
             widx_v, gidx_v, buf0, buf1, ob0, ob1, cvm,
             sem0, sem1, osem0, osem1):
    nc = 2
    wid = lax.axis_index("s") * nc + lax.axis_index("c")
    base = wid * BPW

    pltpu.sync_copy(widx_hbm.at[pl.ds(base * L, BPW * L)], widx_v)
    pltpu.sync_copy(gidx_hbm.at[pl.ds(base * L, BPW * L)], gidx_v)

    def gather(c, buf, sem):
        return pltpu.make_async_copy(
            emb_hbm.at[widx_v.at[pl.ds(c * RL, RL)]], buf, sem)

    def outcp(c, ob, sem):
        return pltpu.make_async_copy(
            ob, wf_hbm.at[pl.ds(base + c * R, R)], sem)

    inv_l = jnp.full((32,), INV_L, jnp.bfloat16)

    def reduce_chunk(buf, ob):
        def row_body(r, _):
            rb = r * L
            acc = [[buf[rb + p, pl.ds(32 * j, 32)] for j in range(2)]
                   for p in range(2)]
            for l in range(2, L):
                p = l % 2
                for j in range(2):
                    acc[p][j] = acc[p][j] + buf[rb + l, pl.ds(32 * j, 32)]
            for j in range(2):
                ob[r, pl.ds(32 * j, 32)] = (acc[0][j] + acc[1][j]) * inv_l
            return 0

        lax.fori_loop(0, R, row_body, 0)

    gather(0, buf0, sem0).start()

    def pair_body(p, _):
        c0 = 2 * p
        gather(c0 + 1, buf1, sem1).start()
        gather(c0, buf0, sem0).wait()

        @pl.when(p > 0)
        def _():
            outcp(c0 - 2, ob0, osem0).wait()

        reduce_chunk(buf0, ob0)
        outcp(c0, ob0, osem0).start()

        @pl.when(p < NCH // 2 - 1)
        def _():
            gather(c0 + 2, buf0, sem0).start()

        gather(c0 + 1, buf1, sem1).wait()

        @pl.when(p > 0)
        def _():
            outcp(c0 - 1, ob1, osem1).wait()

        reduce_chunk(buf1, ob1)
        outcp(c0 + 1, ob1, osem1).start()
        return 0

    lax.fori_loop(0, NCH // 2, pair_body, 0)
    outcp(NCH - 2, ob0, osem0).wait()
    outcp(NCH - 1, ob1, osem1).wait()

    zero16 = jnp.zeros((16,), jnp.float32)
    ones16 = jnp.full((16,), INV_L, jnp.float32)
    iota16 = lax.iota(jnp.int32, 16)

    def geo_group(gi, _):
        for r in range(16):
            cvm[r, pl.ds(0, 16)] = zero16
            cvm[r, pl.ds(16, 16)] = zero16
        gbase = gi * 16 * L
        for l in range(L):
            vals = plsc.load_gather(gidx_v, [gbase + iota16 * L + l])
            plsc.addupdate_scatter(cvm, [iota16, vals], ones16)
        pltpu.sync_copy(cvm, cnt_hbm.at[pl.ds(base + gi * 16, 16)])
        return 0

    lax.fori_loop(0, BPW // 16, geo_group, 0)


_sc_gather = pl.kernel(
    _sc_body,
    out_type=(
        jax.ShapeDtypeStruct((B, D), jnp.bfloat16),
        jax.ShapeDtypeStruct((B, GEO_PAD), jnp.float32),
    ),
    mesh=plsc.VectorSubcoreMesh(core_axis_name="c", subcore_axis_name="s"),
    compiler_params=pltpu.CompilerParams(
        needs_layout_passes=False, use_tc_tiling_on_sc=False),
    scratch_types=[
        pltpu.VMEM((BPW * L,), jnp.int32),
        pltpu.VMEM((BPW * L,), jnp.int32),
        pltpu.VMEM((RL, D), jnp.bfloat16),
        pltpu.VMEM((RL, D), jnp.bfloat16),
        pltpu.VMEM((R, D), jnp.bfloat16),
        pltpu.VMEM((R, D), jnp.bfloat16),
        pltpu.VMEM((16, GEO_PAD), jnp.float32),
        pltpu.SemaphoreType.DMA,
        pltpu.SemaphoreType.DMA,
        pltpu.SemaphoreType.DMA,
        pltpu.SemaphoreType.DMA,
    ],
)


BT = 512


def _const(shape):
    return pl.BlockSpec(shape, lambda i: (0, 0))


def _tiled(shape):
    return pl.BlockSpec(shape, lambda i: (i, 0))


def _tfidf_body(scales_ref, xt_ref, w1_ref, b1_ref, w2_ref, b2_ref, tf_ref):
    f32 = jnp.float32
    bf16 = jnp.bfloat16
    tf_scale = scales_ref[0, 0]
    h = lax.dot_general(xt_ref[...].astype(bf16), w1_ref[...],
                        (((0,), (0,)), ((), ())),
                        preferred_element_type=f32)
    h = h + b1_ref[...]
    h = jnp.dot(h.astype(bf16), w2_ref[...],
                preferred_element_type=f32) + b2_ref[...]
    tf = jax.nn.sigmoid(jnp.clip(h * tf_scale, -10.0, 10.0))
    tf_ref[...] = tf.astype(bf16)


def _tfidf(scales, xt, w1, b1, w2, b2):
    return pl.pallas_call(
        _tfidf_body,
        grid=(B // BT,),
        in_specs=[
            _const((1, 2)),
            pl.BlockSpec((MAXF, BT), lambda i: (0, i)),
            _const((MAXF, D)),
            _const((1, D)),
            _const((D, D)),
            _const((1, D)),
        ],
        out_specs=_tiled((BT, D)),
        out_shape=jax.ShapeDtypeStruct((B, D), jnp.bfloat16),
        compiler_params=pltpu.CompilerParams(
            dimension_semantics=("arbitrary",),
        ),
    )(scales, xt, w1, b1, w2, b2)


VT = 2048


def _tpose_body(xt_ref, o_ref):
    o_ref[...] = xt_ref[...].T.astype(jnp.bfloat16)


def _tpose(wembt):
    nblk = (VOCAB_ROWS + VT - 1) // VT
    return pl.pallas_call(
        _tpose_body,
        grid=(nblk,),
        in_specs=[pl.BlockSpec((D, VT), lambda i: (0, i))],
        out_specs=pl.BlockSpec((VT, D), lambda i: (i, 0)),
        out_shape=jax.ShapeDtypeStruct((VOCAB_ROWS, D), jnp.bfloat16),
        compiler_params=pltpu.CompilerParams(
            dimension_semantics=("arbitrary",),
        ),
    )(wembt)


def _combine_body(scales_ref, tf_ref, wf_ref, cnt_ref, pos_ref, geo_ref,
                  cw1a_ref, cw1b_ref, cw1c_ref, cw1d_ref, cb1_ref,
                  cw2_ref, cb2_ref, o_ref):
    f32 = jnp.float32
    bf16 = jnp.bfloat16
    geo_scale = scales_ref[0, 1]

    pos_row = jnp.sum(pos_ref[...], axis=0, keepdims=True) * INV_L

    c1 = jnp.dot(tf_ref[...], cw1a_ref[...], preferred_element_type=f32)
    c1 = c1 + jnp.dot(wf_ref[...], cw1b_ref[...], preferred_element_type=f32)
    c1 = c1 + jnp.dot(pos_row.astype(bf16), cw1c_ref[...],
                      preferred_element_type=f32)
    geo_comb = jnp.dot(geo_ref[...], cw1d_ref[...], preferred_element_type=f32)
    c1 = c1 + jnp.dot(cnt_ref[...].astype(bf16), geo_comb.astype(bf16),
                      preferred_element_type=f32)
    c1 = c1 + cb1_ref[...]

    c2 = jnp.dot(c1.astype(bf16), cw2_ref[...],
                 preferred_element_type=f32) + cb2_ref[...]
    compass = jax.nn.sigmoid(jnp.clip(c2 * geo_scale, -10.0, 10.0))
    a = compass * (math.pi / 4)
    b = compass * (math.pi / 6)
    a2 = a * a
    b2 = b * b
    sin_a = a * (1.0 + a2 * (-1.0 / 6.0 + a2 * (1.0 / 120.0
                                                + a2 * (-1.0 / 5040.0))))
    cos_b = 1.0 + b2 * (-0.5 + b2 * (1.0 / 24.0 + b2 * (-1.0 / 720.0)))
    o_ref[...] = compass * 0.9 + sin_a * cos_b * 0.1


def _combine(scales, tf, wf, cnt, pos, geo,
             cw1a, cw1b, cw1c, cw1d, cb1, cw2, cb2):
    return pl.pallas_call(
        _combine_body,
        grid=(B // BT,),
        in_specs=[
            _const((1, 2)),
            _tiled((BT, D)),
            _tiled((BT, D)),
            _tiled((BT, GEO_PAD)),
            _const((L, D)),
            _const((GEO_PAD, D)),
            _const((D, NN)),
            _const((D, NN)),
            _const((D, NN)),
            _const((D, NN)),
            _const((1, NN)),
            _const((NN, NN)),
            _const((1, NN)),
        ],
        out_specs=_tiled((BT, NN)),
        out_shape=jax.ShapeDtypeStruct((B, NN), jnp.float32),
        compiler_params=pltpu.CompilerParams(
            dimension_semantics=("arbitrary",),
        ),
    )(scales, tf, wf, cnt, pos, geo,
      cw1a, cw1b, cw1c, cw1d, cb1, cw2, cb2)


def kernel(tfidf_features, word_indices, geo_indices,
           tfidf_W1, tfidf_b1, tfidf_W2, tfidf_b2,
           word_emb, pos_emb, geo_emb,
           comp_W1, comp_b1, comp_W2, comp_b2,
           tfidf_scale, geo_scale):
    bf16 = jnp.bfloat16
    widx = word_indices.reshape(-1).astype(jnp.int32)
    gidx = geo_indices.reshape(-1).astype(jnp.int32)
    wemb_bf = _tpose(word_emb.T)
    wf, cnt = _sc_gather(wemb_bf, widx, gidx)

    scales = jnp.stack([tfidf_scale, geo_scale]).reshape(1, 2).astype(jnp.float32)
    pos = pos_emb[:L]
    geo = geo_emb[:GEO_PAD].astype(bf16)
    cw1a = comp_W1[0:D].astype(bf16)
    cw1b = comp_W1[D:2 * D].astype(bf16)
    cw1c = comp_W1[2 * D:3 * D].astype(bf16)
    cw1d = comp_W1[3 * D:4 * D].astype(bf16)
    tf = _tfidf(scales, tfidf_features.T, tfidf_W1.astype(bf16),
                tfidf_b1.reshape(1, D), tfidf_W2.astype(bf16),
                tfidf_b2.reshape(1, D))
    return _combine(scales, tf, wf, cnt, pos, geo,
                    cw1a, cw1b, cw1c, cw1d, comp_b1.reshape(1, NN),
                    comp_W2.astype(bf16), comp_b2.reshape(1, NN))


# speedup vs baseline: 29.9792x; 1.1367x over previous
import functools
import math

import jax
import jax.numpy as jnp
from jax import lax
from jax.experimental import pallas as pl
from jax.experimental.pallas import tpu as pltpu
from jax.experimental.pallas import tpu_sc as plsc

B = 16384
L = 50
MAXF = 1000
NN = 256
D = NN // 4
GEO_PAD = 32
VOCAB_ROWS = 100001

NW = 32
BPW = B // NW
R = 8
RL = R * L
NCH = BPW // R
INV_L = 1.0 / L


def _sc_body(emb_hbm, widx_hbm, gidx_hbm, wf_hbm, cnt_hbm,
             widx_v, gidx_v, buf0, buf1, ob0, ob1, cvm,
             sem0, sem1, osem0, osem1):
    nc = 2
    wid = lax.axis_index("s") * nc + lax.axis_index("c")
    base = wid * BPW

    pltpu.sync_copy(widx_hbm.at[pl.ds(base * L, BPW * L)], widx_v)
    pltpu.sync_copy(gidx_hbm.at[pl.ds(base * L, BPW * L)], gidx_v)

    def gather(c, buf, sem):
        return pltpu.make_async_copy(
            emb_hbm.at[widx_v.at[pl.ds(c * RL, RL)]], buf, sem)

    def outcp(c, ob, sem):
        return pltpu.make_async_copy(
            ob, wf_hbm.at[pl.ds(base + c * R, R)], sem)

    def reduce_chunk(buf, ob):
        def row_body(r, _):
            rb = r * L
            acc = [[buf[rb + p, pl.ds(16 * j, 16)] for j in range(4)]
                   for p in range(2)]
            for l in range(2, L):
                p = l % 2
                for j in range(4):
                    acc[p][j] = acc[p][j] + buf[rb + l, pl.ds(16 * j, 16)]
            for j in range(4):
                ob[r, pl.ds(16 * j, 16)] = (acc[0][j] + acc[1][j]) * INV_L
            return 0

        lax.fori_loop(0, R, row_body, 0)

    gather(0, buf0, sem0).start()

    def pair_body(p, _):
        c0 = 2 * p
        gather(c0 + 1, buf1, sem1).start()
        gather(c0, buf0, sem0).wait()

        @pl.when(p > 0)
        def _():
            outcp(c0 - 2, ob0, osem0).wait()

        reduce_chunk(buf0, ob0)
        outcp(c0, ob0, osem0).start()

        @pl.when(p < NCH // 2 - 1)
        def _():
            gather(c0 + 2, buf0, sem0).start()

        gather(c0 + 1, buf1, sem1).wait()

        @pl.when(p > 0)
        def _():
            outcp(c0 - 1, ob1, osem1).wait()

        reduce_chunk(buf1, ob1)
        outcp(c0 + 1, ob1, osem1).start()
        return 0

    lax.fori_loop(0, NCH // 2, pair_body, 0)
    outcp(NCH - 2, ob0, osem0).wait()
    outcp(NCH - 1, ob1, osem1).wait()

    zero16 = jnp.zeros((16,), jnp.float32)
    ones16 = jnp.full((16,), INV_L, jnp.float32)
    iota16 = lax.iota(jnp.int32, 16)

    def geo_group(gi, _):
        for r in range(16):
            cvm[r, pl.ds(0, 16)] = zero16
            cvm[r, pl.ds(16, 16)] = zero16
        gbase = gi * 16 * L
        for l in range(L):
            vals = plsc.load_gather(gidx_v, [gbase + iota16 * L + l])
            plsc.addupdate_scatter(cvm, [iota16, vals], ones16)
        pltpu.sync_copy(cvm, cnt_hbm.at[pl.ds(base + gi * 16, 16)])
        return 0

    lax.fori_loop(0, BPW // 16, geo_group, 0)


_sc_gather = pl.kernel(
    _sc_body,
    out_type=(
        jax.ShapeDtypeStruct((B, D), jnp.float32),
        jax.ShapeDtypeStruct((B, GEO_PAD), jnp.float32),
    ),
    mesh=plsc.VectorSubcoreMesh(core_axis_name="c", subcore_axis_name="s"),
    compiler_params=pltpu.CompilerParams(
        needs_layout_passes=False, use_tc_tiling_on_sc=False),
    scratch_types=[
        pltpu.VMEM((BPW * L,), jnp.int32),
        pltpu.VMEM((BPW * L,), jnp.int32),
        pltpu.VMEM((RL, D), jnp.float32),
        pltpu.VMEM((RL, D), jnp.float32),
        pltpu.VMEM((R, D), jnp.float32),
        pltpu.VMEM((R, D), jnp.float32),
        pltpu.VMEM((16, GEO_PAD), jnp.float32),
        pltpu.SemaphoreType.DMA,
        pltpu.SemaphoreType.DMA,
        pltpu.SemaphoreType.DMA,
        pltpu.SemaphoreType.DMA,
    ],
)


BT = 512


def _const(shape):
    return pl.BlockSpec(shape, lambda i: (0, 0))


def _tiled(shape):
    return pl.BlockSpec(shape, lambda i: (i, 0))


def _tfidf_body(scales_ref, xt_ref, w1_ref, b1_ref, w2_ref, b2_ref, tf_ref):
    f32 = jnp.float32
    bf16 = jnp.bfloat16
    tf_scale = scales_ref[0, 0]
    h = lax.dot_general(xt_ref[...].astype(bf16), w1_ref[...],
                        (((0,), (0,)), ((), ())),
                        preferred_element_type=f32)
    h = h + b1_ref[...]
    h = jnp.dot(h.astype(bf16), w2_ref[...],
                preferred_element_type=f32) + b2_ref[...]
    tf = jax.nn.sigmoid(jnp.clip(h * tf_scale, -10.0, 10.0))
    tf_ref[...] = tf.astype(bf16)


def _tfidf(scales, xt, w1, b1, w2, b2):
    return pl.pallas_call(
        _tfidf_body,
        grid=(B // BT,),
        in_specs=[
            _const((1, 2)),
            pl.BlockSpec((MAXF, BT), lambda i: (0, i)),
            _const((MAXF, D)),
            _const((1, D)),
            _const((D, D)),
            _const((1, D)),
        ],
        out_specs=_tiled((BT, D)),
        out_shape=jax.ShapeDtypeStruct((B, D), jnp.bfloat16),
        compiler_params=pltpu.CompilerParams(
            dimension_semantics=("arbitrary",),
        ),
    )(scales, xt, w1, b1, w2, b2)


VT = 2048


def _tpose_body(xt_ref, o_ref):
    o_ref[...] = xt_ref[...].T.astype(jnp.bfloat16)


def _tpose(wembt):
    nblk = (VOCAB_ROWS + VT - 1) // VT
    return pl.pallas_call(
        _tpose_body,
        grid=(nblk,),
        in_specs=[pl.BlockSpec((D, VT), lambda i: (0, i))],
        out_specs=pl.BlockSpec((VT, D), lambda i: (i, 0)),
        out_shape=jax.ShapeDtypeStruct((VOCAB_ROWS, D), jnp.bfloat16),
        compiler_params=pltpu.CompilerParams(
            dimension_semantics=("arbitrary",),
        ),
    )(wembt)


def _combine_body(scales_ref, tf_ref, wf_ref, cnt_ref, pos_ref, geo_ref,
                  cw1a_ref, cw1b_ref, cw1c_ref, cw1d_ref, cb1_ref,
                  cw2_ref, cb2_ref, o_ref):
    f32 = jnp.float32
    bf16 = jnp.bfloat16
    geo_scale = scales_ref[0, 1]

    pos_row = jnp.sum(pos_ref[...], axis=0, keepdims=True) * INV_L

    c1 = jnp.dot(tf_ref[...], cw1a_ref[...], preferred_element_type=f32)
    c1 = c1 + jnp.dot(wf_ref[...].astype(bf16), cw1b_ref[...],
                      preferred_element_type=f32)
    c1 = c1 + jnp.dot(pos_row.astype(bf16), cw1c_ref[...],
                      preferred_element_type=f32)
    geo_comb = jnp.dot(geo_ref[...], cw1d_ref[...], preferred_element_type=f32)
    c1 = c1 + jnp.dot(cnt_ref[...].astype(bf16), geo_comb.astype(bf16),
                      preferred_element_type=f32)
    c1 = c1 + cb1_ref[...]

    c2 = jnp.dot(c1.astype(bf16), cw2_ref[...],
                 preferred_element_type=f32) + cb2_ref[...]
    compass = jax.nn.sigmoid(jnp.clip(c2 * geo_scale, -10.0, 10.0))
    a = compass * (math.pi / 4)
    b = compass * (math.pi / 6)
    a2 = a * a
    b2 = b * b
    sin_a = a * (1.0 + a2 * (-1.0 / 6.0 + a2 * (1.0 / 120.0
                                                + a2 * (-1.0 / 5040.0))))
    cos_b = 1.0 + b2 * (-0.5 + b2 * (1.0 / 24.0 + b2 * (-1.0 / 720.0)))
    o_ref[...] = compass * 0.9 + sin_a * cos_b * 0.1


def _combine(scales, tf, wf, cnt, pos, geo,
             cw1a, cw1b, cw1c, cw1d, cb1, cw2, cb2):
    return pl.pallas_call(
        _combine_body,
        grid=(B // BT,),
        in_specs=[
            _const((1, 2)),
            _tiled((BT, D)),
            _tiled((BT, D)),
            _tiled((BT, GEO_PAD)),
            _const((L, D)),
            _const((GEO_PAD, D)),
            _const((D, NN)),
            _const((D, NN)),
            _const((D, NN)),
            _const((D, NN)),
            _const((1, NN)),
            _const((NN, NN)),
            _const((1, NN)),
        ],
        out_specs=_tiled((BT, NN)),
        out_shape=jax.ShapeDtypeStruct((B, NN), jnp.float32),
        compiler_params=pltpu.CompilerParams(
            dimension_semantics=("arbitrary",),
        ),
    )(scales, tf, wf, cnt, pos, geo,
      cw1a, cw1b, cw1c, cw1d, cb1, cw2, cb2)


def kernel(tfidf_features, word_indices, geo_indices,
           tfidf_W1, tfidf_b1, tfidf_W2, tfidf_b2,
           word_emb, pos_emb, geo_emb,
           comp_W1, comp_b1, comp_W2, comp_b2,
           tfidf_scale, geo_scale):
    bf16 = jnp.bfloat16
    widx = word_indices.reshape(-1).astype(jnp.int32)
    gidx = geo_indices.reshape(-1).astype(jnp.int32)
    wf, cnt = _sc_gather(word_emb, widx, gidx)

    scales = jnp.stack([tfidf_scale, geo_scale]).reshape(1, 2).astype(jnp.float32)
    pos = pos_emb[:L]
    geo = geo_emb[:GEO_PAD].astype(bf16)
    cw1a = comp_W1[0:D].astype(bf16)
    cw1b = comp_W1[D:2 * D].astype(bf16)
    cw1c = comp_W1[2 * D:3 * D].astype(bf16)
    cw1d = comp_W1[3 * D:4 * D].astype(bf16)
    tf = _tfidf(scales, tfidf_features.T, tfidf_W1.astype(bf16),
                tfidf_b1.reshape(1, D), tfidf_W2.astype(bf16),
                tfidf_b2.reshape(1, D))
    return _combine(scales, tf, wf, cnt, pos, geo,
                    cw1a, cw1b, cw1c, cw1d, comp_b1.reshape(1, NN),
                    comp_W2.astype(bf16), comp_b2.reshape(1, NN))


# speedup vs baseline: 30.4286x; 1.0150x over previous
import functools
import math

import jax
import jax.numpy as jnp
from jax import lax
from jax.experimental import pallas as pl
from jax.experimental.pallas import tpu as pltpu
from jax.experimental.pallas import tpu_sc as plsc

B = 16384
L = 50
MAXF = 1000
NN = 256
D = NN // 4
GEO_PAD = 32
VOCAB_ROWS = 100001

NW = 32
BPW = B // NW
R = 8
RL = R * L
NCH = BPW // R
INV_L = 1.0 / L


def _sc_body(emb_hbm, widx_hbm, gidx_hbm, wf_hbm, cnt_hbm,
             widx_v, gidx_v, buf0, buf1, ob0, ob1, cvm,
             sem0, sem1, osem0, osem1):
    nc = 2
    wid = lax.axis_index("s") * nc + lax.axis_index("c")
    base = wid * BPW

    pltpu.sync_copy(widx_hbm.at[pl.ds(base * L, BPW * L)], widx_v)
    pltpu.sync_copy(gidx_hbm.at[pl.ds(base * L, BPW * L)], gidx_v)

    def gather(c, buf, sem):
        return pltpu.make_async_copy(
            emb_hbm.at[widx_v.at[pl.ds(c * RL, RL)]], buf, sem)

    def outcp(c, ob, sem):
        return pltpu.make_async_copy(
            ob, wf_hbm.at[pl.ds(base + c * R, R)], sem)

    def reduce_chunk(buf, ob):
        def row_body(r, _):
            rb = r * L
            acc = [[buf[rb + p, pl.ds(16 * j, 16)] for j in range(4)]
                   for p in range(2)]
            for l in range(2, L):
                p = l % 2
                for j in range(4):
                    acc[p][j] = acc[p][j] + buf[rb + l, pl.ds(16 * j, 16)]
            for j in range(4):
                ob[r, pl.ds(16 * j, 16)] = (acc[0][j] + acc[1][j]) * INV_L
            return 0

        lax.fori_loop(0, R, row_body, 0)

    gather(0, buf0, sem0).start()

    def pair_body(p, _):
        c0 = 2 * p
        gather(c0 + 1, buf1, sem1).start()
        gather(c0, buf0, sem0).wait()

        @pl.when(p > 0)
        def _():
            outcp(c0 - 2, ob0, osem0).wait()

        reduce_chunk(buf0, ob0)
        outcp(c0, ob0, osem0).start()

        @pl.when(p < NCH // 2 - 1)
        def _():
            gather(c0 + 2, buf0, sem0).start()

        gather(c0 + 1, buf1, sem1).wait()

        @pl.when(p > 0)
        def _():
            outcp(c0 - 1, ob1, osem1).wait()

        reduce_chunk(buf1, ob1)
        outcp(c0 + 1, ob1, osem1).start()
        return 0

    lax.fori_loop(0, NCH // 2, pair_body, 0)
    outcp(NCH - 2, ob0, osem0).wait()
    outcp(NCH - 1, ob1, osem1).wait()

    zero16 = jnp.zeros((16,), jnp.float32)
    ones16 = jnp.full((16,), INV_L, jnp.float32)
    iota16 = lax.iota(jnp.int32, 16)

    def geo_group(gi, _):
        for r in range(16):
            cvm[r, pl.ds(0, 16)] = zero16
            cvm[r, pl.ds(16, 16)] = zero16
        gbase = gi * 16 * L
        for l in range(L):
            vals = plsc.load_gather(gidx_v, [gbase + iota16 * L + l])
            plsc.addupdate_scatter(cvm, [iota16, vals], ones16)
        pltpu.sync_copy(cvm, cnt_hbm.at[pl.ds(base + gi * 16, 16)])
        return 0

    lax.fori_loop(0, BPW // 16, geo_group, 0)


_sc_gather = pl.kernel(
    _sc_body,
    out_type=(
        jax.ShapeDtypeStruct((B, D), jnp.float32),
        jax.ShapeDtypeStruct((B, GEO_PAD), jnp.float32),
    ),
    mesh=plsc.VectorSubcoreMesh(core_axis_name="c", subcore_axis_name="s"),
    compiler_params=pltpu.CompilerParams(
        needs_layout_passes=False, use_tc_tiling_on_sc=False),
    scratch_types=[
        pltpu.VMEM((BPW * L,), jnp.int32),
        pltpu.VMEM((BPW * L,), jnp.int32),
        pltpu.VMEM((RL, D), jnp.float32),
        pltpu.VMEM((RL, D), jnp.float32),
        pltpu.VMEM((R, D), jnp.float32),
        pltpu.VMEM((R, D), jnp.float32),
        pltpu.VMEM((16, GEO_PAD), jnp.float32),
        pltpu.SemaphoreType.DMA,
        pltpu.SemaphoreType.DMA,
        pltpu.SemaphoreType.DMA,
        pltpu.SemaphoreType.DMA,
    ],
)


BT = 1024


def _const(shape):
    return pl.BlockSpec(shape, lambda i: (0, 0))


def _tiled(shape):
    return pl.BlockSpec(shape, lambda i: (i, 0))


def _tfidf_body(scales_ref, xt_ref, w1_ref, b1_ref, w2_ref, b2_ref, tf_ref):
    f32 = jnp.float32
    bf16 = jnp.bfloat16
    tf_scale = scales_ref[0, 0]
    h = lax.dot_general(xt_ref[...].astype(bf16), w1_ref[...],
                        (((0,), (0,)), ((), ())),
                        preferred_element_type=f32)
    h = h + b1_ref[...]
    h = jnp.dot(h.astype(bf16), w2_ref[...],
                preferred_element_type=f32) + b2_ref[...]
    tf = jax.nn.sigmoid(jnp.clip(h * tf_scale, -10.0, 10.0))
    tf_ref[...] = tf.astype(bf16)


def _tfidf(scales, xt, w1, b1, w2, b2):
    return pl.pallas_call(
        _tfidf_body,
        grid=(B // BT,),
        in_specs=[
            _const((1, 2)),
            pl.BlockSpec((MAXF, BT), lambda i: (0, i)),
            _const((MAXF, D)),
            _const((1, D)),
            _const((D, D)),
            _const((1, D)),
        ],
        out_specs=_tiled((BT, D)),
        out_shape=jax.ShapeDtypeStruct((B, D), jnp.bfloat16),
        compiler_params=pltpu.CompilerParams(
            dimension_semantics=("arbitrary",),
        ),
    )(scales, xt, w1, b1, w2, b2)


def _combine_body(scales_ref, tf_ref, wf_ref, cnt_ref, pos_ref, geo_ref,
                  cw1a_ref, cw1b_ref, cw1c_ref, cw1d_ref, cb1_ref,
                  cw2_ref, cb2_ref, o_ref):
    f32 = jnp.float32
    bf16 = jnp.bfloat16
    geo_scale = scales_ref[0, 1]

    pos_row = jnp.sum(pos_ref[...], axis=0, keepdims=True) * INV_L

    c1 = jnp.dot(tf_ref[...], cw1a_ref[...], preferred_element_type=f32)
    c1 = c1 + jnp.dot(wf_ref[...].astype(bf16), cw1b_ref[...],
                      preferred_element_type=f32)
    c1 = c1 + jnp.dot(pos_row.astype(bf16), cw1c_ref[...],
                      preferred_element_type=f32)
    geo_comb = jnp.dot(geo_ref[...], cw1d_ref[...], preferred_element_type=f32)
    c1 = c1 + jnp.dot(cnt_ref[...].astype(bf16), geo_comb.astype(bf16),
                      preferred_element_type=f32)
    c1 = c1 + cb1_ref[...]

    c2 = jnp.dot(c1.astype(bf16), cw2_ref[...],
                 preferred_element_type=f32) + cb2_ref[...]
    compass = jax.nn.sigmoid(jnp.clip(c2 * geo_scale, -10.0, 10.0))
    a = compass * (math.pi / 4)
    b = compass * (math.pi / 6)
    a2 = a * a
    b2 = b * b
    sin_a = a * (1.0 + a2 * (-1.0 / 6.0 + a2 * (1.0 / 120.0
                                                + a2 * (-1.0 / 5040.0))))
    cos_b = 1.0 + b2 * (-0.5 + b2 * (1.0 / 24.0 + b2 * (-1.0 / 720.0)))
    o_ref[...] = compass * 0.9 + sin_a * cos_b * 0.1


def _combine(scales, tf, wf, cnt, pos, geo,
             cw1a, cw1b, cw1c, cw1d, cb1, cw2, cb2):
    return pl.pallas_call(
        _combine_body,
        grid=(B // BT,),
        in_specs=[
            _const((1, 2)),
            _tiled((BT, D)),
            _tiled((BT, D)),
            _tiled((BT, GEO_PAD)),
            _const((L, D)),
            _const((GEO_PAD, D)),
            _const((D, NN)),
            _const((D, NN)),
            _const((D, NN)),
            _const((D, NN)),
            _const((1, NN)),
            _const((NN, NN)),
            _const((1, NN)),
        ],
        out_specs=_tiled((BT, NN)),
        out_shape=jax.ShapeDtypeStruct((B, NN), jnp.float32),
        compiler_params=pltpu.CompilerParams(
            dimension_semantics=("arbitrary",),
        ),
    )(scales, tf, wf, cnt, pos, geo,
      cw1a, cw1b, cw1c, cw1d, cb1, cw2, cb2)


def kernel(tfidf_features, word_indices, geo_indices,
           tfidf_W1, tfidf_b1, tfidf_W2, tfidf_b2,
           word_emb, pos_emb, geo_emb,
           comp_W1, comp_b1, comp_W2, comp_b2,
           tfidf_scale, geo_scale):
    bf16 = jnp.bfloat16
    widx = word_indices.reshape(-1).astype(jnp.int32)
    gidx = geo_indices.reshape(-1).astype(jnp.int32)
    wf, cnt = _sc_gather(word_emb, widx, gidx)

    scales = jnp.stack([tfidf_scale, geo_scale]).reshape(1, 2).astype(jnp.float32)
    pos = pos_emb[:L]
    geo = geo_emb[:GEO_PAD].astype(bf16)
    cw1a = comp_W1[0:D].astype(bf16)
    cw1b = comp_W1[D:2 * D].astype(bf16)
    cw1c = comp_W1[2 * D:3 * D].astype(bf16)
    cw1d = comp_W1[3 * D:4 * D].astype(bf16)
    tf = _tfidf(scales, tfidf_features.T, tfidf_W1.astype(bf16),
                tfidf_b1.reshape(1, D), tfidf_W2.astype(bf16),
                tfidf_b2.reshape(1, D))
    return _combine(scales, tf, wf, cnt, pos, geo,
                    cw1a, cw1b, cw1c, cw1d, comp_b1.reshape(1, NN),
                    comp_W2.astype(bf16), comp_b2.reshape(1, NN))
